# baseline (speedup 1.0000x reference)
import functools

import jax
import jax.numpy as jnp
from jax import lax
from jax.experimental import pallas as pl
from jax.experimental.pallas import tpu as pltpu
from jax.experimental.pallas import tpu_sc as plsc

_NC = 2
_NS = 16
_NW = _NC * _NS
_L = 16
_NBUF = 4

_GATHER_DNUMS = lax.GatherDimensionNumbers(
    offset_dims=(), collapsed_slice_dims=(0,), start_index_map=(0,))


def _lane_shuffle(v, perm):
    return lax.gather(v, perm[:, None], _GATHER_DNUMS, (1,),
                      mode=lax.GatherScatterMode.PROMISE_IN_BOUNDS)


def _allreduce_sum16(v):
    lanes = lax.iota(jnp.int32, _L)
    for sh in (8, 4, 2, 1):
        v = v + _lane_shuffle(v, lanes ^ sh)
    return v


def _rsqrt16(x):
    m = x
    s = jnp.full((_L,), 1.0, jnp.float32)
    for k in (32, 16, 8, 4, 2, 1):
        big = m >= jnp.float32(4.0 ** k)
        m = jnp.where(big, m * jnp.float32(4.0 ** -k), m)
        s = jnp.where(big, s * jnp.float32(2.0 ** -k), s)
    for k in (32, 16, 8, 4, 2, 1):
        small = m < jnp.float32(4.0 ** (1 - k))
        m = jnp.where(small, m * jnp.float32(4.0 ** k), m)
        s = jnp.where(small, s * jnp.float32(2.0 ** k), s)
    big = m >= jnp.float32(2.0)
    m = jnp.where(big, m * jnp.float32(0.5), m)
    s = jnp.where(big, s * jnp.float32(0.7071067811865476), s)
    y = jnp.float32(1.0) - jnp.float32(0.27) * (m - jnp.float32(1.0))
    for _ in range(3):
        y = y * (jnp.float32(1.5) - jnp.float32(0.5) * m * y * y)
    return y * s


@functools.lru_cache(maxsize=None)
def _make_sc_kernel(BS, E, S, R):
    B = BS // S
    SW = S // _NW
    CB = SW // R
    CH = B * CB
    ES = E // _L
    inv_e = 1.0 / E
    mesh = plsc.VectorSubcoreMesh(core_axis_name="c", subcore_axis_name="s")

    @functools.partial(
        pl.kernel,
        out_type=jax.ShapeDtypeStruct((BS, E), jnp.float32),
        mesh=mesh,
        scratch_types=[
            pltpu.VMEM((CH, R), jnp.int32),
            pltpu.VMEM((_NBUF, R, E), jnp.float32),
            pltpu.VMEM((SW, E), jnp.float32),
            pltpu.VMEM((E,), jnp.float32),
            pltpu.VMEM((E,), jnp.float32),
        ] + [pltpu.SemaphoreType.DMA] * (2 * _NBUF),
    )
    def k(tid_hbm, tt_hbm, pt_hbm, g_hbm, b_hbm, out_hbm,
          idx_v, rows_v, pos_v, g_v, b_v, *sems):
        gsem = sems[:_NBUF]
        ssem = sems[_NBUF:]
        wid = lax.axis_index("s") * _NC + lax.axis_index("c")
        pltpu.sync_copy(tid_hbm.at[wid], idx_v)
        pltpu.sync_copy(g_hbm, g_v)
        pltpu.sync_copy(b_hbm, b_v)
        pltpu.sync_copy(pt_hbm.at[pl.ds(wid * SW, SW)], pos_v)

        def row0_of(c):
            return (c // CB) * S + wid * SW + lax.rem(c, CB) * R

        def gather_copy(c, p):
            return pltpu.make_async_copy(
                tt_hbm.at[idx_v.at[c]], rows_v.at[p], gsem[p])

        def store_copy(c, p):
            return pltpu.make_async_copy(
                rows_v.at[p], out_hbm.at[pl.ds(row0_of(c), R)], ssem[p])

        gather_copy(0, 0).start()

        def compute(c, p):
            so = lax.rem(c, CB) * R

            zero = jnp.zeros((_L,), jnp.float32)

            @plsc.parallel_loop(0, ES, unroll=4, carry=(zero,) * (2 * R))
            def carry(e, acc):
                su, sq = acc[:R], acc[R:]
                nsu, nsq = [], []
                for r in range(R):
                    x = (rows_v[p, r, pl.ds(e * _L, _L)]
                         + pos_v[so + r, pl.ds(e * _L, _L)])
                    rows_v[p, r, pl.ds(e * _L, _L)] = x
                    nsu.append(su[r] + x)
                    nsq.append(sq[r] + x * x)
                return tuple(nsu) + tuple(nsq)

            lanes = lax.iota(jnp.int32, _L)
            means, vvpack = [], jnp.zeros((_L,), jnp.float32)
            for r in range(R):
                mean_r = _allreduce_sum16(carry[r]) * inv_e
                vv_r = (_allreduce_sum16(carry[R + r]) * inv_e
                        - mean_r * mean_r + 1e-5)
                means.append(mean_r)
                vvpack = jnp.where(lanes == r, vv_r, vvpack)
            ypack = _rsqrt16(vvpack)
            scales = [_lane_shuffle(ypack, jnp.full((_L,), r, jnp.int32))
                      for r in range(R)]

            @plsc.parallel_loop(0, ES, unroll=4)
            def _(e):
                g16 = g_v[pl.ds(e * _L, _L)]
                b16 = b_v[pl.ds(e * _L, _L)]
                for r in range(R):
                    t = rows_v[p, r, pl.ds(e * _L, _L)]
                    a = scales[r] * g16
                    rows_v[p, r, pl.ds(e * _L, _L)] = (t - means[r]) * a + b16

        def body(j, _):
            for b in range(_NBUF):
                c = j * _NBUF + b
                nxt = (b + 1) % _NBUF

                @pl.when((c >= _NBUF - 1) & (c + 1 < CH))
                def _():
                    store_copy(c - (_NBUF - 1), nxt).wait()

                @pl.when(c + 1 < CH)
                def _():
                    gather_copy(c + 1, nxt).start()

                gather_copy(c, b).wait()
                compute(c, b)
                store_copy(c, b).start()
            return 0

        lax.fori_loop(0, CH // _NBUF, body, 0)
        for b in range(_NBUF):
            store_copy(CH - _NBUF + b, b).wait()

    return k


def kernel(token_ids, token_table, pos_table, gamma, beta):
    B, S = token_ids.shape
    V, E = token_table.shape
    BS = B * S
    R = 8
    SW = S // _NW
    CB = SW // R
    tid = (token_ids.astype(jnp.int32)
           .reshape(B, _NW, CB, R)
           .transpose(1, 0, 2, 3)
           .reshape(_NW, B * CB, R))
    k = _make_sc_kernel(BS, E, S, R)
    out = k(tid, token_table, pos_table, gamma, beta)
    return out.reshape(B, S, E)


# speedup vs baseline: 1.9603x; 1.9603x over previous
import functools

import jax
import jax.numpy as jnp
from jax import lax
from jax.experimental import pallas as pl
from jax.experimental.pallas import tpu as pltpu
from jax.experimental.pallas import tpu_sc as plsc

_NC = 2
_NS = 16
_NW = _NC * _NS
_L = 16
_NBUF = 4
_SPC = 2

_GATHER_DNUMS = lax.GatherDimensionNumbers(
    offset_dims=(), collapsed_slice_dims=(0,), start_index_map=(0,))


def _lane_shuffle(v, perm):
    return lax.gather(v, perm[:, None], _GATHER_DNUMS, (1,),
                      mode=lax.GatherScatterMode.PROMISE_IN_BOUNDS)


def _allreduce_sum16(v):
    lanes = lax.iota(jnp.int32, _L)
    for sh in (8, 4, 2, 1):
        v = v + _lane_shuffle(v, lanes ^ sh)
    return v


def _rsqrt16(x):
    m = x
    s = jnp.full((_L,), 1.0, jnp.float32)
    for k in (32, 16, 8, 4, 2, 1):
        big = m >= jnp.float32(4.0 ** k)
        m = jnp.where(big, m * jnp.float32(4.0 ** -k), m)
        s = jnp.where(big, s * jnp.float32(2.0 ** -k), s)
    for k in (32, 16, 8, 4, 2, 1):
        small = m < jnp.float32(4.0 ** (1 - k))
        m = jnp.where(small, m * jnp.float32(4.0 ** k), m)
        s = jnp.where(small, s * jnp.float32(2.0 ** k), s)
    big = m >= jnp.float32(2.0)
    m = jnp.where(big, m * jnp.float32(0.5), m)
    s = jnp.where(big, s * jnp.float32(0.7071067811865476), s)
    y = jnp.float32(1.0) - jnp.float32(0.27) * (m - jnp.float32(1.0))
    for _ in range(3):
        y = y * (jnp.float32(1.5) - jnp.float32(0.5) * m * y * y)
    return y * s


@functools.lru_cache(maxsize=None)
def _make_sc_kernel(BS, E, S):
    B = BS // S
    R = B * _SPC
    SW = S // _NW
    CH = SW // _SPC
    ES = E // _L
    inv_e = 1.0 / E
    mesh = plsc.VectorSubcoreMesh(core_axis_name="c", subcore_axis_name="s")

    @functools.partial(
        pl.kernel,
        out_type=jax.ShapeDtypeStruct((BS, E), jnp.float32),
        mesh=mesh,
        scratch_types=[
            pltpu.VMEM((CH, R), jnp.int32),
            pltpu.VMEM((CH, R), jnp.int32),
            pltpu.VMEM((_NBUF, R, E), jnp.float32),
            pltpu.VMEM((SW, E), jnp.float32),
            pltpu.VMEM((E,), jnp.float32),
            pltpu.VMEM((E,), jnp.float32),
        ] + [pltpu.SemaphoreType.DMA] * (2 * _NBUF),
    )
    def k(tid_hbm, orow_hbm, tt_hbm, pt_hbm, g_hbm, b_hbm, out_hbm,
          idx_v, oidx_v, rows_v, pos_v, g_v, b_v, *sems):
        gsem = sems[:_NBUF]
        ssem = sems[_NBUF:]
        wid = lax.axis_index("s") * _NC + lax.axis_index("c")
        pltpu.sync_copy(tid_hbm.at[wid], idx_v)
        pltpu.sync_copy(orow_hbm.at[wid], oidx_v)
        pltpu.sync_copy(g_hbm, g_v)
        pltpu.sync_copy(b_hbm, b_v)
        pltpu.sync_copy(pt_hbm.at[pl.ds(wid * SW, SW)], pos_v)

        def gather_copy(c, p):
            return pltpu.make_async_copy(
                tt_hbm.at[idx_v.at[c]], rows_v.at[p], gsem[p])

        def store_copy(c, p):
            return pltpu.make_async_copy(
                rows_v.at[p], out_hbm.at[oidx_v.at[c]], ssem[p])

        gather_copy(0, 0).start()
        gather_copy(1, 1).start()

        def compute(c, p):
            so = c * _SPC
            zero = jnp.zeros((_L,), jnp.float32)

            @plsc.parallel_loop(0, ES, unroll=2, carry=(zero,) * (2 * R))
            def carry(e, acc):
                su, sq = acc[:R], acc[R:]
                xs = [rows_v[p, r, pl.ds(e * _L, _L)] for r in range(R)]
                ps = [pos_v[so + si, pl.ds(e * _L, _L)] for si in range(_SPC)]
                nsu, nsq = [], []
                for r in range(R):
                    x = xs[r] + ps[r % _SPC]
                    rows_v[p, r, pl.ds(e * _L, _L)] = x
                    nsu.append(su[r] + x)
                    nsq.append(sq[r] + x * x)
                return tuple(nsu) + tuple(nsq)

            lanes = lax.iota(jnp.int32, _L)
            means, vvpack = [], jnp.zeros((_L,), jnp.float32)
            for r in range(R):
                mean_r = _allreduce_sum16(carry[r]) * inv_e
                vv_r = (_allreduce_sum16(carry[R + r]) * inv_e
                        - mean_r * mean_r + 1e-5)
                means.append(mean_r)
                vvpack = jnp.where(lanes == r, vv_r, vvpack)
            ypack = _rsqrt16(vvpack)
            scales = [_lane_shuffle(ypack, jnp.full((_L,), r, jnp.int32))
                      for r in range(R)]

            @plsc.parallel_loop(0, ES, unroll=2)
            def _(e):
                g16 = g_v[pl.ds(e * _L, _L)]
                b16 = b_v[pl.ds(e * _L, _L)]
                ts = [rows_v[p, r, pl.ds(e * _L, _L)] for r in range(R)]
                for r in range(R):
                    a = scales[r] * g16
                    rows_v[p, r, pl.ds(e * _L, _L)] = (ts[r] - means[r]) * a + b16

        def body(j, _):
            for b in range(_NBUF):
                c = j * _NBUF + b
                nxt2 = (b + 2) % _NBUF

                @pl.when((c >= 2) & (c + 2 < CH))
                def _():
                    store_copy(c - 2, nxt2).wait()

                @pl.when(c + 2 < CH)
                def _():
                    gather_copy(c + 2, nxt2).start()

                gather_copy(c, b).wait()
                compute(c, b)
                store_copy(c, b).start()
            return 0

        lax.fori_loop(0, CH // _NBUF, body, 0)
        for b in range(_NBUF):
            store_copy(CH - _NBUF + b, b).wait()

    return k


def kernel(token_ids, token_table, pos_table, gamma, beta):
    B, S = token_ids.shape
    V, E = token_table.shape
    BS = B * S
    SW = S // _NW
    CH = SW // _SPC
    tid = (token_ids.astype(jnp.int32)
           .reshape(B, _NW, CH, _SPC)
           .transpose(1, 2, 0, 3)
           .reshape(_NW, CH, B * _SPC))
    orow = (
        jnp.arange(B, dtype=jnp.int32)[None, None, :, None] * S
        + jnp.arange(_NW, dtype=jnp.int32)[:, None, None, None] * SW
        + jnp.arange(CH, dtype=jnp.int32)[None, :, None, None] * _SPC
        + jnp.arange(_SPC, dtype=jnp.int32)[None, None, None, :]
    ).reshape(_NW, CH, B * _SPC)
    k = _make_sc_kernel(BS, E, S)
    out = k(tid, orow, token_table, pos_table, gamma, beta)
    return out.reshape(B, S, E)


# speedup vs baseline: 1.9787x; 1.0094x over previous
import functools

import jax
import jax.numpy as jnp
from jax import lax
from jax.experimental import pallas as pl
from jax.experimental.pallas import tpu as pltpu
from jax.experimental.pallas import tpu_sc as plsc

_NC = 2
_NS = 16
_NW = _NC * _NS
_L = 16
_NBUF = 4
_SPC = 2

_GATHER_DNUMS = lax.GatherDimensionNumbers(
    offset_dims=(), collapsed_slice_dims=(0,), start_index_map=(0,))


def _lane_shuffle(v, perm):
    return lax.gather(v, perm[:, None], _GATHER_DNUMS, (1,),
                      mode=lax.GatherScatterMode.PROMISE_IN_BOUNDS)


def _allreduce_sum16(v):
    lanes = lax.iota(jnp.int32, _L)
    for sh in (8, 4, 2, 1):
        v = v + _lane_shuffle(v, lanes ^ sh)
    return v


def _rsqrt16(x):
    m = x
    s = jnp.full((_L,), 1.0, jnp.float32)
    for k in (32, 16, 8, 4, 2, 1):
        big = m >= jnp.float32(4.0 ** k)
        m = jnp.where(big, m * jnp.float32(4.0 ** -k), m)
        s = jnp.where(big, s * jnp.float32(2.0 ** -k), s)
    for k in (32, 16, 8, 4, 2, 1):
        small = m < jnp.float32(4.0 ** (1 - k))
        m = jnp.where(small, m * jnp.float32(4.0 ** k), m)
        s = jnp.where(small, s * jnp.float32(2.0 ** k), s)
    big = m >= jnp.float32(2.0)
    m = jnp.where(big, m * jnp.float32(0.5), m)
    s = jnp.where(big, s * jnp.float32(0.7071067811865476), s)
    y = jnp.float32(1.0) - jnp.float32(0.27) * (m - jnp.float32(1.0))
    for _ in range(3):
        y = y * (jnp.float32(1.5) - jnp.float32(0.5) * m * y * y)
    return y * s


@functools.lru_cache(maxsize=None)
def _make_sc_kernel(BS, E, S):
    B = BS // S
    R = B * _SPC
    SW = S // _NW
    CH = SW // _SPC
    ES = E // _L
    inv_e = 1.0 / E
    mesh = plsc.VectorSubcoreMesh(core_axis_name="c", subcore_axis_name="s")

    @functools.partial(
        pl.kernel,
        out_type=jax.ShapeDtypeStruct((BS, E), jnp.float32),
        mesh=mesh,
        scratch_types=[
            pltpu.VMEM((CH, R), jnp.int32),
            pltpu.VMEM((CH, R), jnp.int32),
            pltpu.VMEM((_NBUF, R, E), jnp.float32),
            pltpu.VMEM((SW, E), jnp.float32),
            pltpu.VMEM((E,), jnp.float32),
            pltpu.VMEM((E,), jnp.float32),
        ] + [pltpu.SemaphoreType.DMA] * (2 * _NBUF),
    )
    def k(tid_hbm, orow_hbm, tt_hbm, pt_hbm, g_hbm, b_hbm, out_hbm,
          idx_v, oidx_v, rows_v, pos_v, g_v, b_v, *sems):
        gsem = sems[:_NBUF]
        ssem = sems[_NBUF:]
        wid = lax.axis_index("s") * _NC + lax.axis_index("c")
        pltpu.sync_copy(tid_hbm.at[wid], idx_v)
        pltpu.sync_copy(orow_hbm.at[wid], oidx_v)
        pltpu.sync_copy(g_hbm, g_v)
        pltpu.sync_copy(b_hbm, b_v)
        pltpu.sync_copy(pt_hbm.at[pl.ds(wid * SW, SW)], pos_v)

        def gather_copy(c, p):
            return pltpu.make_async_copy(
                tt_hbm.at[idx_v.at[c]], rows_v.at[p], gsem[p])

        def store_copy(c, p):
            return pltpu.make_async_copy(
                rows_v.at[p], out_hbm.at[oidx_v.at[c]], ssem[p])

        gather_copy(0, 0).start()
        gather_copy(1, 1).start()

        def compute(c, p):
            so = c * _SPC
            zero = jnp.zeros((_L,), jnp.float32)

            @plsc.parallel_loop(0, ES, unroll=2, carry=(zero,) * (2 * R))
            def carry(e, acc):
                su, sq = acc[:R], acc[R:]
                xs = [rows_v[p, r, pl.ds(e * _L, _L)] for r in range(R)]
                ps = [pos_v[so + si, pl.ds(e * _L, _L)] for si in range(_SPC)]
                nsu, nsq = [], []
                for r in range(R):
                    x = xs[r] + ps[r % _SPC]
                    nsu.append(su[r] + x)
                    nsq.append(sq[r] + x * x)
                return tuple(nsu) + tuple(nsq)

            lanes = lax.iota(jnp.int32, _L)
            means, vvpack = [], jnp.zeros((_L,), jnp.float32)
            for r in range(R):
                mean_r = _allreduce_sum16(carry[r]) * inv_e
                vv_r = (_allreduce_sum16(carry[R + r]) * inv_e
                        - mean_r * mean_r + 1e-5)
                means.append(mean_r)
                vvpack = jnp.where(lanes == r, vv_r, vvpack)
            ypack = _rsqrt16(vvpack)
            scales = [_lane_shuffle(ypack, jnp.full((_L,), r, jnp.int32))
                      for r in range(R)]

            @plsc.parallel_loop(0, ES, unroll=2)
            def _(e):
                g16 = g_v[pl.ds(e * _L, _L)]
                b16 = b_v[pl.ds(e * _L, _L)]
                ts = [rows_v[p, r, pl.ds(e * _L, _L)] for r in range(R)]
                ps = [pos_v[so + si, pl.ds(e * _L, _L)] for si in range(_SPC)]
                for r in range(R):
                    a = scales[r] * g16
                    t = ts[r] + ps[r % _SPC]
                    rows_v[p, r, pl.ds(e * _L, _L)] = (t - means[r]) * a + b16

        def body(j, _):
            for b in range(_NBUF):
                c = j * _NBUF + b
                nxt2 = (b + 2) % _NBUF

                @pl.when((c >= 2) & (c + 2 < CH))
                def _():
                    store_copy(c - 2, nxt2).wait()

                @pl.when(c + 2 < CH)
                def _():
                    gather_copy(c + 2, nxt2).start()

                gather_copy(c, b).wait()
                compute(c, b)
                store_copy(c, b).start()
            return 0

        lax.fori_loop(0, CH // _NBUF, body, 0)
        for b in range(_NBUF):
            store_copy(CH - _NBUF + b, b).wait()

    return k


def kernel(token_ids, token_table, pos_table, gamma, beta):
    B, S = token_ids.shape
    V, E = token_table.shape
    BS = B * S
    SW = S // _NW
    CH = SW // _SPC
    tid = (token_ids.astype(jnp.int32)
           .reshape(B, _NW, CH, _SPC)
           .transpose(1, 2, 0, 3)
           .reshape(_NW, CH, B * _SPC))
    orow = (
        jnp.arange(B, dtype=jnp.int32)[None, None, :, None] * S
        + jnp.arange(_NW, dtype=jnp.int32)[:, None, None, None] * SW
        + jnp.arange(CH, dtype=jnp.int32)[None, :, None, None] * _SPC
        + jnp.arange(_SPC, dtype=jnp.int32)[None, None, None, :]
    ).reshape(_NW, CH, B * _SPC)
    k = _make_sc_kernel(BS, E, S)
    out = k(tid, orow, token_table, pos_table, gamma, beta)
    return out.reshape(B, S, E)


# speedup vs baseline: 2.0513x; 1.0367x over previous
import functools

import jax
import jax.numpy as jnp
from jax import lax
from jax.experimental import pallas as pl
from jax.experimental.pallas import tpu as pltpu
from jax.experimental.pallas import tpu_sc as plsc

_NC = 2
_NS = 16
_NW = _NC * _NS
_L = 16
_NBUF = 4
_SPC = 2

_GATHER_DNUMS = lax.GatherDimensionNumbers(
    offset_dims=(), collapsed_slice_dims=(0,), start_index_map=(0,))


def _lane_shuffle(v, perm):
    return lax.gather(v, perm[:, None], _GATHER_DNUMS, (1,),
                      mode=lax.GatherScatterMode.PROMISE_IN_BOUNDS)


def _allreduce_sum16(v):
    lanes = lax.iota(jnp.int32, _L)
    for sh in (8, 4, 2, 1):
        v = v + _lane_shuffle(v, lanes ^ sh)
    return v


def _rsqrt16(x):
    m = x
    s = jnp.full((_L,), 1.0, jnp.float32)
    for k in (32, 16, 8, 4, 2, 1):
        big = m >= jnp.float32(4.0 ** k)
        m = jnp.where(big, m * jnp.float32(4.0 ** -k), m)
        s = jnp.where(big, s * jnp.float32(2.0 ** -k), s)
    for k in (32, 16, 8, 4, 2, 1):
        small = m < jnp.float32(4.0 ** (1 - k))
        m = jnp.where(small, m * jnp.float32(4.0 ** k), m)
        s = jnp.where(small, s * jnp.float32(2.0 ** k), s)
    big = m >= jnp.float32(2.0)
    m = jnp.where(big, m * jnp.float32(0.5), m)
    s = jnp.where(big, s * jnp.float32(0.7071067811865476), s)
    y = jnp.float32(1.0) - jnp.float32(0.27) * (m - jnp.float32(1.0))
    for _ in range(3):
        y = y * (jnp.float32(1.5) - jnp.float32(0.5) * m * y * y)
    return y * s


@functools.lru_cache(maxsize=None)
def _make_sc_kernel(BS, E, S):
    B = BS // S
    R = B * _SPC
    SW = S // _NW
    CH = SW // _SPC
    ES = E // _L
    inv_e = 1.0 / E
    mesh = plsc.VectorSubcoreMesh(core_axis_name="c", subcore_axis_name="s")

    @functools.partial(
        pl.kernel,
        out_type=jax.ShapeDtypeStruct((BS, E), jnp.float32),
        mesh=mesh,
        scratch_types=[
            pltpu.VMEM((CH, R), jnp.int32),
            pltpu.VMEM((CH, R), jnp.int32),
            pltpu.VMEM((_NBUF, R, E), jnp.float32),
            pltpu.VMEM((SW, E), jnp.float32),
            pltpu.VMEM((E,), jnp.float32),
            pltpu.VMEM((E,), jnp.float32),
        ] + [pltpu.SemaphoreType.DMA] * (2 * _NBUF + 1),
    )
    def k(tid_hbm, orow_hbm, tt_hbm, pt_hbm, g_hbm, b_hbm, out_hbm,
          idx_v, oidx_v, rows_v, pos_v, g_v, b_v, *sems):
        gsem = sems[:_NBUF]
        ssem = sems[_NBUF:2 * _NBUF]
        psem = sems[2 * _NBUF]
        wid = lax.axis_index("s") * _NC + lax.axis_index("c")
        pltpu.sync_copy(tid_hbm.at[wid], idx_v)

        def gather_copy(c, p):
            return pltpu.make_async_copy(
                tt_hbm.at[idx_v.at[c]], rows_v.at[p], gsem[p])

        def store_copy(c, p):
            return pltpu.make_async_copy(
                rows_v.at[p], out_hbm.at[oidx_v.at[c]], ssem[p])

        pos_copy = pltpu.make_async_copy(
            pt_hbm.at[pl.ds(wid * SW, SW)], pos_v, psem)

        gather_copy(0, 0).start()
        gather_copy(1, 1).start()
        pos_copy.start()
        pltpu.sync_copy(orow_hbm.at[wid], oidx_v)
        pltpu.sync_copy(g_hbm, g_v)
        pltpu.sync_copy(b_hbm, b_v)
        pos_copy.wait()

        def compute(c, p):
            so = c * _SPC
            zero = jnp.zeros((_L,), jnp.float32)

            @plsc.parallel_loop(0, ES, unroll=2, carry=(zero,) * (2 * R))
            def carry(e, acc):
                su, sq = acc[:R], acc[R:]
                xs = [rows_v[p, r, pl.ds(e * _L, _L)] for r in range(R)]
                ps = [pos_v[so + si, pl.ds(e * _L, _L)] for si in range(_SPC)]
                nsu, nsq = [], []
                for r in range(R):
                    x = xs[r] + ps[r % _SPC]
                    nsu.append(su[r] + x)
                    nsq.append(sq[r] + x * x)
                return tuple(nsu) + tuple(nsq)

            lanes = lax.iota(jnp.int32, _L)
            means, vvpack = [], jnp.zeros((_L,), jnp.float32)
            for r in range(R):
                mean_r = _allreduce_sum16(carry[r]) * inv_e
                vv_r = (_allreduce_sum16(carry[R + r]) * inv_e
                        - mean_r * mean_r + 1e-5)
                means.append(mean_r)
                vvpack = jnp.where(lanes == r, vv_r, vvpack)
            ypack = _rsqrt16(vvpack)
            scales = [_lane_shuffle(ypack, jnp.full((_L,), r, jnp.int32))
                      for r in range(R)]

            @plsc.parallel_loop(0, ES, unroll=2)
            def _(e):
                g16 = g_v[pl.ds(e * _L, _L)]
                b16 = b_v[pl.ds(e * _L, _L)]
                ts = [rows_v[p, r, pl.ds(e * _L, _L)] for r in range(R)]
                ps = [pos_v[so + si, pl.ds(e * _L, _L)] for si in range(_SPC)]
                for r in range(R):
                    a = scales[r] * g16
                    t = ts[r] + ps[r % _SPC]
                    rows_v[p, r, pl.ds(e * _L, _L)] = (t - means[r]) * a + b16

        def body(j, _):
            for b in range(_NBUF):
                c = j * _NBUF + b
                nxt2 = (b + 2) % _NBUF

                @pl.when((c >= 2) & (c + 2 < CH))
                def _():
                    store_copy(c - 2, nxt2).wait()

                @pl.when(c + 2 < CH)
                def _():
                    gather_copy(c + 2, nxt2).start()

                gather_copy(c, b).wait()
                compute(c, b)
                store_copy(c, b).start()
            return 0

        lax.fori_loop(0, CH // _NBUF, body, 0)
        for b in range(_NBUF):
            store_copy(CH - _NBUF + b, b).wait()

    return k


def kernel(token_ids, token_table, pos_table, gamma, beta):
    B, S = token_ids.shape
    V, E = token_table.shape
    BS = B * S
    SW = S // _NW
    CH = SW // _SPC
    tid = (token_ids.astype(jnp.int32)
           .reshape(B, _NW, CH, _SPC)
           .transpose(1, 2, 0, 3)
           .reshape(_NW, CH, B * _SPC))
    orow = (
        jnp.arange(B, dtype=jnp.int32)[None, None, :, None] * S
        + jnp.arange(_NW, dtype=jnp.int32)[:, None, None, None] * SW
        + jnp.arange(CH, dtype=jnp.int32)[None, :, None, None] * _SPC
        + jnp.arange(_SPC, dtype=jnp.int32)[None, None, None, :]
    ).reshape(_NW, CH, B * _SPC)
    k = _make_sc_kernel(BS, E, S)
    out = k(tid, orow, token_table, pos_table, gamma, beta)
    return out.reshape(B, S, E)


# speedup vs baseline: 2.0853x; 1.0166x over previous
import functools

import jax
import jax.numpy as jnp
from jax import lax
from jax.experimental import pallas as pl
from jax.experimental.pallas import tpu as pltpu
from jax.experimental.pallas import tpu_sc as plsc

_NC = 2
_NS = 16
_NW = _NC * _NS
_L = 16
_NBUF = 4
_SPC = 2

_GATHER_DNUMS = lax.GatherDimensionNumbers(
    offset_dims=(), collapsed_slice_dims=(0,), start_index_map=(0,))


def _lane_shuffle(v, perm):
    return lax.gather(v, perm[:, None], _GATHER_DNUMS, (1,),
                      mode=lax.GatherScatterMode.PROMISE_IN_BOUNDS)


def _allreduce_sum16(v):
    lanes = lax.iota(jnp.int32, _L)
    for sh in (8, 4, 2, 1):
        v = v + _lane_shuffle(v, lanes ^ sh)
    return v


def _rsqrt16(x):
    m = x
    s = jnp.full((_L,), 1.0, jnp.float32)
    for k in (32, 16, 8, 4, 2, 1):
        big = m >= jnp.float32(4.0 ** k)
        m = jnp.where(big, m * jnp.float32(4.0 ** -k), m)
        s = jnp.where(big, s * jnp.float32(2.0 ** -k), s)
    for k in (32, 16, 8, 4, 2, 1):
        small = m < jnp.float32(4.0 ** (1 - k))
        m = jnp.where(small, m * jnp.float32(4.0 ** k), m)
        s = jnp.where(small, s * jnp.float32(2.0 ** k), s)
    big = m >= jnp.float32(2.0)
    m = jnp.where(big, m * jnp.float32(0.5), m)
    s = jnp.where(big, s * jnp.float32(0.7071067811865476), s)
    y = jnp.float32(1.0) - jnp.float32(0.27) * (m - jnp.float32(1.0))
    for _ in range(3):
        y = y * (jnp.float32(1.5) - jnp.float32(0.5) * m * y * y)
    return y * s


@functools.lru_cache(maxsize=None)
def _make_sc_kernel(BS, E, S):
    B = BS // S
    R = B * _SPC
    SW = S // _NW
    CH = SW // _SPC
    ES = E // _L
    inv_e = 1.0 / E
    mesh = plsc.VectorSubcoreMesh(core_axis_name="c", subcore_axis_name="s")

    @functools.partial(
        pl.kernel,
        out_type=jax.ShapeDtypeStruct((BS, E), jnp.float32),
        mesh=mesh,
        scratch_types=[
            pltpu.VMEM((CH, R), jnp.int32),
            pltpu.VMEM((CH, R), jnp.int32),
            pltpu.VMEM((_NBUF, R, E), jnp.float32),
            pltpu.VMEM((SW, E), jnp.float32),
            pltpu.VMEM((E,), jnp.float32),
            pltpu.VMEM((E,), jnp.float32),
        ] + [pltpu.SemaphoreType.DMA] * (2 * _NBUF + 1),
    )
    def k(tid_hbm, orow_hbm, tt_hbm, pt_hbm, g_hbm, b_hbm, out_hbm,
          idx_v, oidx_v, rows_v, pos_v, g_v, b_v, *sems):
        gsem = sems[:_NBUF]
        ssem = sems[_NBUF:2 * _NBUF]
        psem = sems[2 * _NBUF]
        wid = lax.axis_index("s") * _NC + lax.axis_index("c")
        pltpu.sync_copy(tid_hbm.at[wid], idx_v)

        def gather_copy(c, p):
            return pltpu.make_async_copy(
                tt_hbm.at[idx_v.at[c]], rows_v.at[p], gsem[p])

        def store_copy(c, p):
            return pltpu.make_async_copy(
                rows_v.at[p], out_hbm.at[oidx_v.at[c]], ssem[p])

        pos_copy = pltpu.make_async_copy(
            pt_hbm.at[pl.ds(wid * SW, SW)], pos_v, psem)

        gather_copy(0, 0).start()
        gather_copy(1, 1).start()
        pos_copy.start()
        pltpu.sync_copy(orow_hbm.at[wid], oidx_v)
        pltpu.sync_copy(g_hbm, g_v)
        pltpu.sync_copy(b_hbm, b_v)
        pos_copy.wait()

        def compute(c, p):
            so = c * _SPC
            zero = jnp.zeros((_L,), jnp.float32)

            @plsc.parallel_loop(0, ES, unroll=2, carry=(zero,) * (2 * R))
            def carry(e, acc):
                su, sq = acc[:R], acc[R:]
                xs = [rows_v[p, r, pl.ds(e * _L, _L)] for r in range(R)]
                ps = [pos_v[so + si, pl.ds(e * _L, _L)] for si in range(_SPC)]
                nsu, nsq = [], []
                for r in range(R):
                    x = xs[r] + ps[r % _SPC]
                    nsu.append(su[r] + x)
                    nsq.append(sq[r] + x * x)
                return tuple(nsu) + tuple(nsq)

            lanes = lax.iota(jnp.int32, _L)
            vecs = list(carry)
            for sh in (8, 4, 2, 1):
                nv = []
                for i in range(len(vecs) // 2):
                    a, b = vecs[2 * i], vecs[2 * i + 1]
                    pa = a + _lane_shuffle(a, lanes ^ sh)
                    pb = b + _lane_shuffle(b, lanes ^ sh)
                    nv.append(jnp.where((lanes & sh) == 0, pa, pb))
                vecs = nv
            tot = vecs[0]
            su_perm = (((lanes & 1) << 3) | ((lanes & 2) << 1)
                       | ((lanes & 4) >> 1))
            meanpack = _lane_shuffle(tot, su_perm) * inv_e
            sqpack = _lane_shuffle(tot, su_perm + 1) * inv_e
            vv = sqpack - meanpack * meanpack + 1e-5
            ypack = _rsqrt16(vv)
            means = [_lane_shuffle(meanpack, jnp.full((_L,), r, jnp.int32))
                     for r in range(R)]
            scales = [_lane_shuffle(ypack, jnp.full((_L,), r, jnp.int32))
                      for r in range(R)]

            @plsc.parallel_loop(0, ES, unroll=2)
            def _(e):
                g16 = g_v[pl.ds(e * _L, _L)]
                b16 = b_v[pl.ds(e * _L, _L)]
                ts = [rows_v[p, r, pl.ds(e * _L, _L)] for r in range(R)]
                ps = [pos_v[so + si, pl.ds(e * _L, _L)] for si in range(_SPC)]
                for r in range(R):
                    a = scales[r] * g16
                    t = ts[r] + ps[r % _SPC]
                    rows_v[p, r, pl.ds(e * _L, _L)] = (t - means[r]) * a + b16

        def body(j, _):
            for b in range(_NBUF):
                c = j * _NBUF + b
                nxt2 = (b + 2) % _NBUF

                @pl.when((c >= 2) & (c + 2 < CH))
                def _():
                    store_copy(c - 2, nxt2).wait()

                @pl.when(c + 2 < CH)
                def _():
                    gather_copy(c + 2, nxt2).start()

                gather_copy(c, b).wait()
                compute(c, b)
                store_copy(c, b).start()
            return 0

        lax.fori_loop(0, CH // _NBUF, body, 0)
        for b in range(_NBUF):
            store_copy(CH - _NBUF + b, b).wait()

    return k


def kernel(token_ids, token_table, pos_table, gamma, beta):
    B, S = token_ids.shape
    V, E = token_table.shape
    BS = B * S
    SW = S // _NW
    CH = SW // _SPC
    tid = (token_ids.astype(jnp.int32)
           .reshape(B, _NW, CH, _SPC)
           .transpose(1, 2, 0, 3)
           .reshape(_NW, CH, B * _SPC))
    orow = (
        jnp.arange(B, dtype=jnp.int32)[None, None, :, None] * S
        + jnp.arange(_NW, dtype=jnp.int32)[:, None, None, None] * SW
        + jnp.arange(CH, dtype=jnp.int32)[None, :, None, None] * _SPC
        + jnp.arange(_SPC, dtype=jnp.int32)[None, None, None, :]
    ).reshape(_NW, CH, B * _SPC)
    k = _make_sc_kernel(BS, E, S)
    out = k(tid, orow, token_table, pos_table, gamma, beta)
    return out.reshape(B, S, E)
